# baseline (speedup 1.0000x reference)
import functools

import jax
import jax.numpy as jnp
import numpy as np
from jax import lax
from jax.experimental import pallas as pl
from jax.experimental.pallas import tpu as pltpu
from jax.experimental.pallas import tpu_sc as plsc

NC = 2
NS = 16
NW = NC * NS


def _bn(h, g, b):
    m = jnp.mean(h, axis=0, keepdims=True)
    v = jnp.mean((h - m) * (h - m), axis=0, keepdims=True)
    return (h - m) * lax.rsqrt(v + 1e-5) * g + b


def _embed_body(x_ref, w_ref, b_ref, g_ref, bb_ref, o_ref):
    h = jnp.dot(x_ref[...], w_ref[...], preferred_element_type=jnp.float32)
    h = h + b_ref[...]
    o_ref[...] = jnp.maximum(_bn(h, g_ref[...], bb_ref[...]), 0.0)


def _tc_embed(x, w, b, g, bb):
    n = x.shape[0]
    dout = w.shape[1]
    return pl.pallas_call(
        _embed_body,
        out_shape=jax.ShapeDtypeStruct((n, dout), jnp.float32),
    )(x, w, b.reshape(1, -1), g.reshape(1, -1), bb.reshape(1, -1))


def _block_body(h_ref, p0_ref, p1_ref, w1_ref, b1_ref, w2_ref, b2_ref,
                g_ref, bb_ref, o_ref):
    h = h_ref[...]
    s = p0_ref[...] + p1_ref[...]
    din = h.shape[1]
    w1 = w1_ref[...]
    w2 = w2_ref[...]
    a = (jnp.dot(h, w1[:din], preferred_element_type=jnp.float32)
         + jnp.dot(s, w1[din:], preferred_element_type=jnp.float32)
         + b1_ref[...])
    c = (jnp.dot(h, w2[:din], preferred_element_type=jnp.float32)
         + jnp.dot(s, w2[din:], preferred_element_type=jnp.float32)
         + b2_ref[...])
    hc = jnp.concatenate([jnp.maximum(a, 0.0), c], axis=1)
    o_ref[...] = _bn(hc, g_ref[...], bb_ref[...])


def _tc_block(h, p0, p1, w1, b1, w2, b2, g, bb):
    n = h.shape[0]
    dout = g.shape[0]
    return pl.pallas_call(
        _block_body,
        out_shape=jax.ShapeDtypeStruct((n, dout), jnp.float32),
    )(h, p0, p1, w1, b1.reshape(1, -1), w2, b2.reshape(1, -1),
      g.reshape(1, -1), bb.reshape(1, -1))


def _block_fin_body(h_ref, p0_ref, p1_ref, w1_ref, b1_ref, w2_ref, b2_ref,
                    g_ref, bb_ref, fw_ref, fb_ref, o_ref, of_ref):
    h = h_ref[...]
    s = jnp.concatenate([p0_ref[...], p1_ref[...]], axis=1)
    din = h.shape[1]
    w1 = w1_ref[...]
    w2 = w2_ref[...]
    a = (jnp.dot(h, w1[:din], preferred_element_type=jnp.float32)
         + jnp.dot(s, w1[din:], preferred_element_type=jnp.float32)
         + b1_ref[...])
    c = (jnp.dot(h, w2[:din], preferred_element_type=jnp.float32)
         + jnp.dot(s, w2[din:], preferred_element_type=jnp.float32)
         + b2_ref[...])
    hc = jnp.concatenate([jnp.maximum(a, 0.0), c], axis=1)
    hb = _bn(hc, g_ref[...], bb_ref[...])
    o_ref[...] = hb.astype(jnp.bfloat16)
    of_ref[...] = jnp.tanh(
        jnp.dot(hb, fw_ref[...], preferred_element_type=jnp.float32)
        + fb_ref[...])


def _tc_block_final(h, p0, p1, w1, b1, w2, b2, g, bb, fw, fb):
    n = h.shape[0]
    dout = g.shape[0]
    return pl.pallas_call(
        _block_fin_body,
        out_shape=(jax.ShapeDtypeStruct((n, dout), jnp.bfloat16),
                   jax.ShapeDtypeStruct((n, fw.shape[1]), jnp.float32)),
    )(h, p0, p1, w1, b1.reshape(1, -1), w2, b2.reshape(1, -1),
      g.reshape(1, -1), bb.reshape(1, -1), fw, fb.reshape(1, -1))


def _edge_mlp_body(e_ref, w1_ref, b1_ref, w2_ref, b2_ref, o_ref):
    t = jnp.maximum(
        jnp.dot(e_ref[...].astype(jnp.float32), w1_ref[...],
                preferred_element_type=jnp.float32)
        + b1_ref[...], 0.0)
    w = jnp.sum(t * w2_ref[...], axis=1) + b2_ref[0, 0]
    o_ref[...] = jax.nn.sigmoid(w).reshape(1, 1, -1)


def _tc_edge_mlp(e, w1, b1, w2, b2):
    E = e.shape[0]
    che = 4000
    nblk = E // che
    out = pl.pallas_call(
        _edge_mlp_body,
        grid=(nblk,),
        in_specs=[
            pl.BlockSpec((che, e.shape[1]), lambda i: (i, 0)),
            pl.BlockSpec(w1.shape, lambda i: (0, 0)),
            pl.BlockSpec((1, w1.shape[1]), lambda i: (0, 0)),
            pl.BlockSpec((1, w1.shape[1]), lambda i: (0, 0)),
            pl.BlockSpec((1, 1), lambda i: (0, 0)),
        ],
        out_specs=pl.BlockSpec((1, 1, che), lambda i: (i, 0, 0)),
        out_shape=jax.ShapeDtypeStruct((nblk, 1, che), jnp.float32),
    )(e, w1, b1.reshape(1, -1), w2.reshape(1, -1), b2.reshape(1, 1))
    return out.reshape(E)


def _sc_spmm(h, src, dst2, w):
    n, din = h.shape
    E = src.shape[0]
    epw = E // NW
    ch = 40
    nch = epw // ch
    nbuf = 5
    nit = nch // (2 * nbuf)
    zch = 80
    nrc = n // zch
    nf = din // 16
    mesh = plsc.VectorSubcoreMesh(core_axis_name="c", subcore_axis_name="s",
                                  num_cores=NC, num_subcores=NS)

    scratch = ([pltpu.VMEM((epw,), jnp.int32),
                pltpu.VMEM((nch, ch), jnp.int32),
                pltpu.VMEM((epw + 8,), jnp.float32)]
               + [pltpu.VMEM((ch, din), jnp.float32) for _ in range(2 * nbuf)]
               + [pltpu.VMEM((zch, din), jnp.float32),
                  pltpu.VMEM_SHARED((n, din), jnp.float32)]
               + [pltpu.SemaphoreType.DMA] * 4)

    @functools.partial(
        pl.kernel,
        out_type=jax.ShapeDtypeStruct((NC, n, din), jnp.float32),
        mesh=mesh,
        scratch_types=scratch,
        compiler_params=pltpu.CompilerParams(use_tc_tiling_on_sc=False),
    )
    def k(h_hbm, src_hbm, dst_hbm, w_hbm, out_hbm, src_v, dst_v, w_v, *rest):
        rows = rest[:2 * nbuf]
        zz_v, acc_sh, gsA, gsB, ssA, ssB = rest[2 * nbuf:]
        c = lax.axis_index("c")
        s = lax.axis_index("s")
        wid = c * NS + s
        base = wid * epw
        pltpu.async_copy(src_hbm.at[pl.ds(base, epw)], src_v, gsA)
        pltpu.async_copy(dst_hbm.at[wid], dst_v, gsA)
        pltpu.async_copy(w_hbm.at[pl.ds(base, epw)], w_v.at[pl.ds(0, epw)],
                         gsA)

        def zrow(r, carry):
            for f in range(nf):
                zz_v[r, pl.ds(f * 16, 16)] = jnp.zeros((16,), jnp.float32)
            return carry
        lax.fori_loop(0, zch, zrow, 0)
        cnt = (nrc - s + NS - 1) // NS
        def zchunk(i, carry):
            pltpu.async_copy(zz_v, acc_sh.at[pl.ds((s + i * NS) * zch, zch)],
                             ssA)
            return carry
        lax.fori_loop(0, cnt, zchunk, 0)
        pltpu.make_async_copy(src_hbm.at[pl.ds(base, epw)], src_v, gsA).wait()
        pltpu.make_async_copy(dst_hbm.at[wid], dst_v, gsA).wait()
        pltpu.make_async_copy(w_hbm.at[pl.ds(base, epw)],
                              w_v.at[pl.ds(0, epw)], gsA).wait()
        def zdrain(i, carry):
            pltpu.make_async_copy(zz_v, acc_sh.at[pl.ds(0, zch)], ssA).wait()
            return carry
        lax.fori_loop(0, cnt, zdrain, 0)
        plsc.subcore_barrier()

        def gather(j, buf, sem):
            pltpu.async_copy(h_hbm.at[src_v.at[pl.ds(j * ch, ch)]], buf, sem)

        def process_half(it, half):
            bufs = rows[half * nbuf:(half + 1) * nbuf]
            gsem = gsA if half == 0 else gsB
            ssem = ssA if half == 0 else ssB
            cbase = it * 2 * nbuf + half * nbuf
            for b in range(nbuf):
                pltpu.make_async_copy(
                    h_hbm.at[src_v.at[pl.ds(0, ch)]], bufs[b], gsem).wait()
            for b in range(nbuf):
                j = cbase + b
                buf = bufs[b]

                def g8(g, cc, _buf=buf, _j=j):
                    wvec = w_v[pl.ds(_j * ch + g * 8, 16)]
                    for e8 in range(8):
                        wb = jnp.full((16,), wvec[e8], jnp.float32)
                        for f in range(nf):
                            _buf[g * 8 + e8, pl.ds(f * 16, 16)] = (
                                _buf[g * 8 + e8, pl.ds(f * 16, 16)] * wb)
                    return cc
                lax.fori_loop(0, ch // 8, g8, 0)
                pltpu.async_copy(buf, acc_sh.at[dst_v.at[j]], ssem, add=True)
            for b in range(nbuf):
                pltpu.make_async_copy(
                    bufs[b], acc_sh.at[dst_v.at[0]], ssem).wait()

            @pl.when(it < nit - 1)
            def _():
                for b in range(nbuf):
                    gather((it + 1) * 2 * nbuf + half * nbuf + b,
                           bufs[b], gsem)

        for half in range(2):
            for b in range(nbuf):
                gather(half * nbuf + b, rows[half * nbuf + b],
                       gsA if half == 0 else gsB)

        def body(it, carry):
            process_half(it, 0)
            process_half(it, 1)
            return carry
        lax.fori_loop(0, nit, body, 0)
        plsc.subcore_barrier()

        def ochunk(i, carry):
            k2 = s + i * NS
            pltpu.async_copy(acc_sh.at[pl.ds(k2 * zch, zch)],
                             out_hbm.at[c, pl.ds(k2 * zch, zch)], ssA)
            return carry
        lax.fori_loop(0, cnt, ochunk, 0)
        def odrain(i, carry):
            pltpu.make_async_copy(acc_sh.at[pl.ds(0, zch)],
                                  out_hbm.at[c, pl.ds(0, zch)], ssA).wait()
            return carry
        lax.fori_loop(0, cnt, odrain, 0)

    return k(h, src, dst2, w)


def _sc_spmm_split(h2, src, dst2s, w, n):
    dh = h2.shape[1]
    E = src.shape[0]
    epc = E // NS
    ch = 40
    nch = epc // ch
    nbuf = 5
    nit = nch // (2 * nbuf)
    zch = 40
    nrc = n // zch
    nf = dh // 16
    mesh = plsc.VectorSubcoreMesh(core_axis_name="c", subcore_axis_name="s",
                                  num_cores=NC, num_subcores=NS)

    scratch = ([pltpu.VMEM((epc,), jnp.int32),
                pltpu.VMEM((nch, ch), jnp.int32),
                pltpu.VMEM((epc + 8,), jnp.float32)]
               + [pltpu.VMEM((ch, dh), jnp.float32) for _ in range(2 * nbuf)]
               + [pltpu.VMEM((zch, dh), jnp.float32),
                  pltpu.VMEM_SHARED((n, dh), jnp.float32)]
               + [pltpu.SemaphoreType.DMA] * 4)

    @functools.partial(
        pl.kernel,
        out_type=jax.ShapeDtypeStruct((NC, n, dh), jnp.float32),
        mesh=mesh,
        scratch_types=scratch,
        compiler_params=pltpu.CompilerParams(use_tc_tiling_on_sc=False),
    )
    def k(h_hbm, src_hbm, dst_hbm, w_hbm, out_hbm, src_v, dst_v, w_v, *rest):
        rows = rest[:2 * nbuf]
        zz_v, acc_sh, gsA, gsB, ssA, ssB = rest[2 * nbuf:]
        c = lax.axis_index("c")
        s = lax.axis_index("s")
        base = s * epc
        pltpu.async_copy(src_hbm.at[pl.ds(base, epc)], src_v, gsA)
        pltpu.async_copy(dst_hbm.at[s], dst_v, gsA)
        pltpu.async_copy(w_hbm.at[pl.ds(base, epc)], w_v.at[pl.ds(0, epc)],
                         gsA)

        def zrow(r, cc):
            for f in range(nf):
                zz_v[r, pl.ds(f * 16, 16)] = jnp.zeros((16,), jnp.float32)
            return cc
        lax.fori_loop(0, zch, zrow, 0)
        cnt = (nrc - s + NS - 1) // NS
        def zchunk(i, cc):
            pltpu.async_copy(zz_v, acc_sh.at[pl.ds((s + i * NS) * zch, zch)],
                             ssA)
            return cc
        lax.fori_loop(0, cnt, zchunk, 0)
        pltpu.make_async_copy(src_hbm.at[pl.ds(base, epc)], src_v, gsA).wait()
        pltpu.make_async_copy(dst_hbm.at[s], dst_v, gsA).wait()
        pltpu.make_async_copy(w_hbm.at[pl.ds(base, epc)],
                              w_v.at[pl.ds(0, epc)], gsA).wait()

        cvec = jnp.full((16,), c, jnp.int32)

        def ixf(g, cc):
            sl = pl.ds(g * 16, 16)
            v = src_v[sl]
            src_v[sl] = v + v + cvec
            return cc
        lax.fori_loop(0, epc // 16, ixf, 0)

        def zdrain(i, cc):
            pltpu.make_async_copy(zz_v, acc_sh.at[pl.ds(0, zch)], ssA).wait()
            return cc
        lax.fori_loop(0, cnt, zdrain, 0)
        plsc.subcore_barrier()

        def gather(j, buf, sem):
            pltpu.async_copy(h_hbm.at[src_v.at[pl.ds(j * ch, ch)]], buf, sem)

        def process_half(it, half):
            bufs = rows[half * nbuf:(half + 1) * nbuf]
            gsem = gsA if half == 0 else gsB
            ssem = ssA if half == 0 else ssB
            cbase = it * 2 * nbuf + half * nbuf
            for b in range(nbuf):
                pltpu.make_async_copy(
                    h_hbm.at[src_v.at[pl.ds(0, ch)]], bufs[b], gsem).wait()
            for b in range(nbuf):
                j = cbase + b
                buf = bufs[b]

                def g8(g, cc, _buf=buf, _j=j):
                    wvec = w_v[pl.ds(_j * ch + g * 8, 16)]
                    for e8 in range(8):
                        wb = jnp.full((16,), wvec[e8], jnp.float32)
                        for f in range(nf):
                            _buf[g * 8 + e8, pl.ds(f * 16, 16)] = (
                                _buf[g * 8 + e8, pl.ds(f * 16, 16)] * wb)
                    return cc
                lax.fori_loop(0, ch // 8, g8, 0)
                pltpu.async_copy(buf, acc_sh.at[dst_v.at[j]], ssem, add=True)
            for b in range(nbuf):
                pltpu.make_async_copy(
                    bufs[b], acc_sh.at[dst_v.at[0]], ssem).wait()

            @pl.when(it < nit - 1)
            def _():
                for b in range(nbuf):
                    gather((it + 1) * 2 * nbuf + half * nbuf + b,
                           bufs[b], gsem)

        for half in range(2):
            for b in range(nbuf):
                gather(half * nbuf + b, rows[half * nbuf + b],
                       gsA if half == 0 else gsB)

        def body(it, carry):
            process_half(it, 0)
            process_half(it, 1)
            return carry
        lax.fori_loop(0, nit, body, 0)
        plsc.subcore_barrier()

        def ochunk(i, carry):
            k2 = s + i * NS
            pltpu.async_copy(acc_sh.at[pl.ds(k2 * zch, zch)],
                             out_hbm.at[c, pl.ds(k2 * zch, zch)], ssA)
            return carry
        lax.fori_loop(0, cnt, ochunk, 0)
        def odrain(i, carry):
            pltpu.make_async_copy(acc_sh.at[pl.ds(0, zch)],
                                  out_hbm.at[c, pl.ds(0, zch)], ssA).wait()
            return carry
        lax.fori_loop(0, cnt, odrain, 0)

    return k(h2, src, dst2s, w)


def _sc_edge_absdiff(h, src, dst):
    n, din = h.shape
    E = src.shape[0]
    epw = E // NW
    ch = 40
    nch = epw // ch
    nbuf = 5
    nit = nch // (2 * nbuf)
    rem = nch - nit * 2 * nbuf
    assert rem in (0, nbuf)
    nf = din // 32
    mesh = plsc.VectorSubcoreMesh(core_axis_name="c", subcore_axis_name="s",
                                  num_cores=NC, num_subcores=NS)

    scratch = ([pltpu.VMEM((epw,), jnp.int32),
                pltpu.VMEM((epw,), jnp.int32)]
               + [pltpu.VMEM((ch, din), jnp.bfloat16)
                  for _ in range(4 * nbuf)]
               + [pltpu.VMEM((ch, din), jnp.float32)
                  for _ in range(2 * nbuf)]
               + [pltpu.SemaphoreType.DMA] * 4)

    @functools.partial(
        pl.kernel,
        out_type=jax.ShapeDtypeStruct((E, din), jnp.float32),
        mesh=mesh,
        scratch_types=scratch,
        compiler_params=pltpu.CompilerParams(use_tc_tiling_on_sc=False,
                                             needs_layout_passes=False),
    )
    def k(h_hbm, src_hbm, dst_hbm, e_hbm, src_v, dst_v, *rest):
        hs = rest[:2 * nbuf]
        hd = rest[2 * nbuf:4 * nbuf]
        ob = rest[4 * nbuf:6 * nbuf]
        gsA, gsB, ssA, ssB = rest[6 * nbuf:]
        c = lax.axis_index("c")
        s = lax.axis_index("s")
        wid = c * NS + s
        base = wid * epw
        pltpu.sync_copy(src_hbm.at[pl.ds(base, epw)], src_v)
        pltpu.sync_copy(dst_hbm.at[pl.ds(base, epw)], dst_v)

        def gather2(j, bs, bd, sem):
            pltpu.async_copy(h_hbm.at[src_v.at[pl.ds(j * ch, ch)]], bs, sem)
            pltpu.async_copy(h_hbm.at[dst_v.at[pl.ds(j * ch, ch)]], bd, sem)

        def process(cbase, bss, bds, obs, gsem, ssem, reissue):
            for b in range(nbuf):
                pltpu.make_async_copy(
                    h_hbm.at[src_v.at[pl.ds(0, ch)]], bss[b], gsem).wait()
                pltpu.make_async_copy(
                    h_hbm.at[src_v.at[pl.ds(0, ch)]], bds[b], gsem).wait()
            for b in range(nbuf):
                j = cbase + b
                bs = bss[b]
                bd = bds[b]
                o = obs[b]

                def ediff(e, cc, _bs=bs, _bd=bd, _o=o):
                    for f in range(nf):
                        sl = pl.ds(f * 32, 32)
                        sa, sb = plsc.unpack(
                            _bs[e, sl], format=plsc.PackFormat.INTERLEAVED)
                        da, db = plsc.unpack(
                            _bd[e, sl], format=plsc.PackFormat.INTERLEAVED)
                        _o[e, pl.ds(f * 32, 16)] = jnp.abs(sa - da)
                        _o[e, pl.ds(f * 32 + 16, 16)] = jnp.abs(sb - db)
                    return cc
                lax.fori_loop(0, ch, ediff, 0)
                pltpu.async_copy(o, e_hbm.at[pl.ds(base + j * ch, ch)], ssem)
            for b in range(nbuf):
                pltpu.make_async_copy(
                    obs[b], e_hbm.at[pl.ds(base, ch)], ssem).wait()
            if reissue is not None:
                for b in range(nbuf):
                    gather2(reissue + b, bss[b], bds[b], gsem)

        bA, bAd, oA = hs[:nbuf], hd[:nbuf], ob[:nbuf]
        bB, bBd, oB = hs[nbuf:], hd[nbuf:], ob[nbuf:]
        for b in range(nbuf):
            gather2(b, bA[b], bAd[b], gsA)
            gather2(nbuf + b, bB[b], bBd[b], gsB)

        def body(it, carry):
            cb = it * 2 * nbuf
            if rem:
                process(cb, bA, bAd, oA, gsA, ssA, cb + 2 * nbuf)
            else:
                process(cb, bA, bAd, oA, gsA, ssA, None)

                @pl.when(it < nit - 1)
                def _():
                    for b in range(nbuf):
                        gather2(cb + 2 * nbuf + b, bA[b], bAd[b], gsA)
            process(cb + nbuf, bB, bBd, oB, gsB, ssB, None)

            @pl.when(it < nit - 1)
            def _():
                for b in range(nbuf):
                    gather2(cb + 3 * nbuf + b, bB[b], bBd[b], gsB)
            return carry
        lax.fori_loop(0, nit, body, 0)
        if rem:
            process(nit * 2 * nbuf, bA, bAd, oA, gsA, ssA, None)

    return k(h, src, dst)


def kernel(x, edge_index, edge_weight, g_size, emb_W, emb_b, bn_g, bn_b,
           blk1_W1, blk1_b1, blk1_W2, blk1_b2, blk1_g, blk1_bb,
           blk2_W1, blk2_b1, blk2_W2, blk2_b2, blk2_g, blk2_bb,
           blk3_W1, blk3_b1, blk3_W2, blk3_b2, blk3_g, blk3_bb,
           blk4_W1, blk4_b1, blk4_W2, blk4_b2, blk4_g, blk4_bb,
           wc_W1, wc_b1, wc_W2, wc_b2, fc_W, fc_b):
    E = edge_index.shape[1]
    n = x.shape[0]
    src = edge_index[0]
    dst = edge_index[1]
    dst2 = dst.reshape(NW, E // NW // 40, 40)
    dst2s = dst.reshape(NS, E // NS // 40, 40)

    h = _tc_embed(x, emb_W, emb_b, bn_g, bn_b)

    p = _sc_spmm(h, src, dst2, edge_weight)
    h = _tc_block(h, p[0], p[1], blk1_W1, blk1_b1, blk1_W2, blk1_b2,
                  blk1_g, blk1_bb)
    p = _sc_spmm(h, src, dst2, edge_weight)
    h = _tc_block(h, p[0], p[1], blk2_W1, blk2_b1, blk2_W2, blk2_b2,
                  blk2_g, blk2_bb)
    p = _sc_spmm(h, src, dst2, edge_weight)
    h = _tc_block(h, p[0], p[1], blk3_W1, blk3_b1, blk3_W2, blk3_b2,
                  blk3_g, blk3_bb)
    p = _sc_spmm_split(h.reshape(2 * n, 64), src, dst2s, edge_weight, n)
    h, out = _tc_block_final(h, p[0], p[1], blk4_W1, blk4_b1, blk4_W2,
                             blk4_b2, blk4_g, blk4_bb, fc_W, fc_b)

    eh = E // 2
    e0 = _sc_edge_absdiff(h, src[:eh], dst[:eh])
    e1 = _sc_edge_absdiff(h, src[eh:], dst[eh:])
    perm = np.concatenate(
        [np.concatenate([np.arange(k, k + 32, 2), np.arange(k + 1, k + 32, 2)])
         for k in range(0, wc_W1.shape[0], 32)])
    w1p = wc_W1[perm]
    m0 = _tc_edge_mlp(e0, w1p, wc_b1, wc_W2, wc_b2)
    m1 = _tc_edge_mlp(e1, w1p, wc_b1, wc_W2, wc_b2)
    wnew = jnp.concatenate([m0, m1])
    return (out, wnew, g_size)


# speedup vs baseline: 1.1162x; 1.1162x over previous
import functools

import jax
import jax.numpy as jnp
from jax import lax
from jax.experimental import pallas as pl
from jax.experimental.pallas import tpu as pltpu
from jax.experimental.pallas import tpu_sc as plsc

NC = 2
NS = 16
NW = NC * NS


def _bn(h, g, b):
    m = jnp.mean(h, axis=0, keepdims=True)
    v = jnp.mean((h - m) * (h - m), axis=0, keepdims=True)
    return (h - m) * lax.rsqrt(v + 1e-5) * g + b


def _embed_body(x_ref, w_ref, b_ref, g_ref, bb_ref, o_ref):
    h = jnp.dot(x_ref[...], w_ref[...], preferred_element_type=jnp.float32)
    h = h + b_ref[...]
    o_ref[...] = jnp.maximum(_bn(h, g_ref[...], bb_ref[...]), 0.0)


def _tc_embed(x, w, b, g, bb):
    n = x.shape[0]
    dout = w.shape[1]
    return pl.pallas_call(
        _embed_body,
        out_shape=jax.ShapeDtypeStruct((n, dout), jnp.float32),
    )(x, w, b.reshape(1, -1), g.reshape(1, -1), bb.reshape(1, -1))


def _block_body(h_ref, p0_ref, p1_ref, w1_ref, b1_ref, w2_ref, b2_ref,
                g_ref, bb_ref, o_ref):
    h = h_ref[...]
    s = p0_ref[...] + p1_ref[...]
    din = h.shape[1]
    w1 = w1_ref[...]
    w2 = w2_ref[...]
    a = (jnp.dot(h, w1[:din], preferred_element_type=jnp.float32)
         + jnp.dot(s, w1[din:], preferred_element_type=jnp.float32)
         + b1_ref[...])
    c = (jnp.dot(h, w2[:din], preferred_element_type=jnp.float32)
         + jnp.dot(s, w2[din:], preferred_element_type=jnp.float32)
         + b2_ref[...])
    hc = jnp.concatenate([jnp.maximum(a, 0.0), c], axis=1)
    o_ref[...] = _bn(hc, g_ref[...], bb_ref[...])


def _tc_block(h, p0, p1, w1, b1, w2, b2, g, bb):
    n = h.shape[0]
    dout = g.shape[0]
    return pl.pallas_call(
        _block_body,
        out_shape=jax.ShapeDtypeStruct((n, dout), jnp.float32),
    )(h, p0, p1, w1, b1.reshape(1, -1), w2, b2.reshape(1, -1),
      g.reshape(1, -1), bb.reshape(1, -1))


def _block_fin_body(h_ref, p0_ref, p1_ref, w1_ref, b1_ref, w2_ref, b2_ref,
                    g_ref, bb_ref, fw_ref, fb_ref, o_ref, of_ref):
    h = h_ref[...]
    s = jnp.concatenate([p0_ref[...], p1_ref[...]], axis=1)
    din = h.shape[1]
    w1 = w1_ref[...]
    w2 = w2_ref[...]
    a = (jnp.dot(h, w1[:din], preferred_element_type=jnp.float32)
         + jnp.dot(s, w1[din:], preferred_element_type=jnp.float32)
         + b1_ref[...])
    c = (jnp.dot(h, w2[:din], preferred_element_type=jnp.float32)
         + jnp.dot(s, w2[din:], preferred_element_type=jnp.float32)
         + b2_ref[...])
    hc = jnp.concatenate([jnp.maximum(a, 0.0), c], axis=1)
    hb = _bn(hc, g_ref[...], bb_ref[...])
    o_ref[...] = hb
    of_ref[...] = jnp.tanh(
        jnp.dot(hb, fw_ref[...], preferred_element_type=jnp.float32)
        + fb_ref[...])


def _tc_block_final(h, p0, p1, w1, b1, w2, b2, g, bb, fw, fb):
    n = h.shape[0]
    dout = g.shape[0]
    return pl.pallas_call(
        _block_fin_body,
        out_shape=(jax.ShapeDtypeStruct((n, dout), jnp.float32),
                   jax.ShapeDtypeStruct((n, fw.shape[1]), jnp.float32)),
    )(h, p0, p1, w1, b1.reshape(1, -1), w2, b2.reshape(1, -1),
      g.reshape(1, -1), bb.reshape(1, -1), fw, fb.reshape(1, -1))


def _edge_mlp_body(e_ref, w1_ref, b1_ref, w2_ref, b2_ref, o_ref):
    t = jnp.maximum(
        jnp.dot(e_ref[...].astype(jnp.float32), w1_ref[...],
                preferred_element_type=jnp.float32)
        + b1_ref[...], 0.0)
    w = jnp.sum(t * w2_ref[...], axis=1) + b2_ref[0, 0]
    o_ref[...] = jax.nn.sigmoid(w).reshape(1, 1, -1)


def _tc_edge_mlp(e, w1, b1, w2, b2):
    E = e.shape[0]
    che = 4000
    nblk = E // che
    out = pl.pallas_call(
        _edge_mlp_body,
        grid=(nblk,),
        in_specs=[
            pl.BlockSpec((che, e.shape[1]), lambda i: (i, 0)),
            pl.BlockSpec(w1.shape, lambda i: (0, 0)),
            pl.BlockSpec((1, w1.shape[1]), lambda i: (0, 0)),
            pl.BlockSpec((1, w1.shape[1]), lambda i: (0, 0)),
            pl.BlockSpec((1, 1), lambda i: (0, 0)),
        ],
        out_specs=pl.BlockSpec((1, 1, che), lambda i: (i, 0, 0)),
        out_shape=jax.ShapeDtypeStruct((nblk, 1, che), jnp.float32),
    )(e, w1, b1.reshape(1, -1), w2.reshape(1, -1), b2.reshape(1, 1))
    return out.reshape(E)


def _sc_spmm(h, src, dst2, w):
    n, din = h.shape
    E = src.shape[0]
    epw = E // NW
    ch = 40
    nch = epw // ch
    nbuf = 5
    nit = nch // (2 * nbuf)
    zch = 80
    nrc = n // zch
    nf = din // 16
    mesh = plsc.VectorSubcoreMesh(core_axis_name="c", subcore_axis_name="s",
                                  num_cores=NC, num_subcores=NS)

    scratch = ([pltpu.VMEM((epw,), jnp.int32),
                pltpu.VMEM((nch, ch), jnp.int32),
                pltpu.VMEM((epw + 8,), jnp.float32)]
               + [pltpu.VMEM((ch, din), jnp.float32) for _ in range(2 * nbuf)]
               + [pltpu.VMEM((zch, din), jnp.float32),
                  pltpu.VMEM_SHARED((n, din), jnp.float32)]
               + [pltpu.SemaphoreType.DMA] * 4)

    @functools.partial(
        pl.kernel,
        out_type=jax.ShapeDtypeStruct((NC, n, din), jnp.float32),
        mesh=mesh,
        scratch_types=scratch,
        compiler_params=pltpu.CompilerParams(use_tc_tiling_on_sc=False),
    )
    def k(h_hbm, src_hbm, dst_hbm, w_hbm, out_hbm, src_v, dst_v, w_v, *rest):
        rows = rest[:2 * nbuf]
        zz_v, acc_sh, gsA, gsB, ssA, ssB = rest[2 * nbuf:]
        c = lax.axis_index("c")
        s = lax.axis_index("s")
        wid = c * NS + s
        base = wid * epw
        pltpu.async_copy(src_hbm.at[pl.ds(base, epw)], src_v, gsA)
        pltpu.async_copy(dst_hbm.at[wid], dst_v, gsA)
        pltpu.async_copy(w_hbm.at[pl.ds(base, epw)], w_v.at[pl.ds(0, epw)],
                         gsA)

        def zrow(r, carry):
            for f in range(nf):
                zz_v[r, pl.ds(f * 16, 16)] = jnp.zeros((16,), jnp.float32)
            return carry
        lax.fori_loop(0, zch, zrow, 0)
        cnt = (nrc - s + NS - 1) // NS
        def zchunk(i, carry):
            pltpu.async_copy(zz_v, acc_sh.at[pl.ds((s + i * NS) * zch, zch)],
                             ssA)
            return carry
        lax.fori_loop(0, cnt, zchunk, 0)
        pltpu.make_async_copy(src_hbm.at[pl.ds(base, epw)], src_v, gsA).wait()
        pltpu.make_async_copy(dst_hbm.at[wid], dst_v, gsA).wait()
        pltpu.make_async_copy(w_hbm.at[pl.ds(base, epw)],
                              w_v.at[pl.ds(0, epw)], gsA).wait()
        def zdrain(i, carry):
            pltpu.make_async_copy(zz_v, acc_sh.at[pl.ds(0, zch)], ssA).wait()
            return carry
        lax.fori_loop(0, cnt, zdrain, 0)
        plsc.subcore_barrier()

        def gather(j, buf, sem):
            pltpu.async_copy(h_hbm.at[src_v.at[pl.ds(j * ch, ch)]], buf, sem)

        def process_half(it, half):
            bufs = rows[half * nbuf:(half + 1) * nbuf]
            gsem = gsA if half == 0 else gsB
            ssem = ssA if half == 0 else ssB
            cbase = it * 2 * nbuf + half * nbuf
            for b in range(nbuf):
                pltpu.make_async_copy(
                    h_hbm.at[src_v.at[pl.ds(0, ch)]], bufs[b], gsem).wait()
            for b in range(nbuf):
                j = cbase + b
                buf = bufs[b]

                def g8(g, cc, _buf=buf, _j=j):
                    wvec = w_v[pl.ds(_j * ch + g * 8, 16)]
                    for e8 in range(8):
                        wb = jnp.full((16,), wvec[e8], jnp.float32)
                        for f in range(nf):
                            _buf[g * 8 + e8, pl.ds(f * 16, 16)] = (
                                _buf[g * 8 + e8, pl.ds(f * 16, 16)] * wb)
                    return cc
                lax.fori_loop(0, ch // 8, g8, 0)
                pltpu.async_copy(buf, acc_sh.at[dst_v.at[j]], ssem, add=True)
            for b in range(nbuf):
                pltpu.make_async_copy(
                    bufs[b], acc_sh.at[dst_v.at[0]], ssem).wait()

            @pl.when(it < nit - 1)
            def _():
                for b in range(nbuf):
                    gather((it + 1) * 2 * nbuf + half * nbuf + b,
                           bufs[b], gsem)

        for half in range(2):
            for b in range(nbuf):
                gather(half * nbuf + b, rows[half * nbuf + b],
                       gsA if half == 0 else gsB)

        def body(it, carry):
            process_half(it, 0)
            process_half(it, 1)
            return carry
        lax.fori_loop(0, nit, body, 0)
        plsc.subcore_barrier()

        def ochunk(i, carry):
            k2 = s + i * NS
            pltpu.async_copy(acc_sh.at[pl.ds(k2 * zch, zch)],
                             out_hbm.at[c, pl.ds(k2 * zch, zch)], ssA)
            return carry
        lax.fori_loop(0, cnt, ochunk, 0)
        def odrain(i, carry):
            pltpu.make_async_copy(acc_sh.at[pl.ds(0, zch)],
                                  out_hbm.at[c, pl.ds(0, zch)], ssA).wait()
            return carry
        lax.fori_loop(0, cnt, odrain, 0)

    return k(h, src, dst2, w)


def _sc_spmm_split(h2, src, dst2s, w, n):
    dh = h2.shape[1]
    E = src.shape[0]
    epc = E // NS
    ch = 40
    nch = epc // ch
    nbuf = 5
    nit = nch // (2 * nbuf)
    zch = 40
    nrc = n // zch
    nf = dh // 16
    mesh = plsc.VectorSubcoreMesh(core_axis_name="c", subcore_axis_name="s",
                                  num_cores=NC, num_subcores=NS)

    scratch = ([pltpu.VMEM((epc,), jnp.int32),
                pltpu.VMEM((nch, ch), jnp.int32),
                pltpu.VMEM((epc + 8,), jnp.float32)]
               + [pltpu.VMEM((ch, dh), jnp.float32) for _ in range(2 * nbuf)]
               + [pltpu.VMEM((zch, dh), jnp.float32),
                  pltpu.VMEM_SHARED((n, dh), jnp.float32)]
               + [pltpu.SemaphoreType.DMA] * 4)

    @functools.partial(
        pl.kernel,
        out_type=jax.ShapeDtypeStruct((NC, n, dh), jnp.float32),
        mesh=mesh,
        scratch_types=scratch,
        compiler_params=pltpu.CompilerParams(use_tc_tiling_on_sc=False),
    )
    def k(h_hbm, src_hbm, dst_hbm, w_hbm, out_hbm, src_v, dst_v, w_v, *rest):
        rows = rest[:2 * nbuf]
        zz_v, acc_sh, gsA, gsB, ssA, ssB = rest[2 * nbuf:]
        c = lax.axis_index("c")
        s = lax.axis_index("s")
        base = s * epc
        pltpu.async_copy(src_hbm.at[pl.ds(base, epc)], src_v, gsA)
        pltpu.async_copy(dst_hbm.at[s], dst_v, gsA)
        pltpu.async_copy(w_hbm.at[pl.ds(base, epc)], w_v.at[pl.ds(0, epc)],
                         gsA)

        def zrow(r, cc):
            for f in range(nf):
                zz_v[r, pl.ds(f * 16, 16)] = jnp.zeros((16,), jnp.float32)
            return cc
        lax.fori_loop(0, zch, zrow, 0)
        cnt = (nrc - s + NS - 1) // NS
        def zchunk(i, cc):
            pltpu.async_copy(zz_v, acc_sh.at[pl.ds((s + i * NS) * zch, zch)],
                             ssA)
            return cc
        lax.fori_loop(0, cnt, zchunk, 0)
        pltpu.make_async_copy(src_hbm.at[pl.ds(base, epc)], src_v, gsA).wait()
        pltpu.make_async_copy(dst_hbm.at[s], dst_v, gsA).wait()
        pltpu.make_async_copy(w_hbm.at[pl.ds(base, epc)],
                              w_v.at[pl.ds(0, epc)], gsA).wait()

        cvec = jnp.full((16,), c, jnp.int32)

        def ixf(g, cc):
            sl = pl.ds(g * 16, 16)
            v = src_v[sl]
            src_v[sl] = v + v + cvec
            return cc
        lax.fori_loop(0, epc // 16, ixf, 0)

        def zdrain(i, cc):
            pltpu.make_async_copy(zz_v, acc_sh.at[pl.ds(0, zch)], ssA).wait()
            return cc
        lax.fori_loop(0, cnt, zdrain, 0)
        plsc.subcore_barrier()

        def gather(j, buf, sem):
            pltpu.async_copy(h_hbm.at[src_v.at[pl.ds(j * ch, ch)]], buf, sem)

        def process_half(it, half):
            bufs = rows[half * nbuf:(half + 1) * nbuf]
            gsem = gsA if half == 0 else gsB
            ssem = ssA if half == 0 else ssB
            cbase = it * 2 * nbuf + half * nbuf
            for b in range(nbuf):
                pltpu.make_async_copy(
                    h_hbm.at[src_v.at[pl.ds(0, ch)]], bufs[b], gsem).wait()
            for b in range(nbuf):
                j = cbase + b
                buf = bufs[b]

                def g8(g, cc, _buf=buf, _j=j):
                    wvec = w_v[pl.ds(_j * ch + g * 8, 16)]
                    for e8 in range(8):
                        wb = jnp.full((16,), wvec[e8], jnp.float32)
                        for f in range(nf):
                            _buf[g * 8 + e8, pl.ds(f * 16, 16)] = (
                                _buf[g * 8 + e8, pl.ds(f * 16, 16)] * wb)
                    return cc
                lax.fori_loop(0, ch // 8, g8, 0)
                pltpu.async_copy(buf, acc_sh.at[dst_v.at[j]], ssem, add=True)
            for b in range(nbuf):
                pltpu.make_async_copy(
                    bufs[b], acc_sh.at[dst_v.at[0]], ssem).wait()

            @pl.when(it < nit - 1)
            def _():
                for b in range(nbuf):
                    gather((it + 1) * 2 * nbuf + half * nbuf + b,
                           bufs[b], gsem)

        for half in range(2):
            for b in range(nbuf):
                gather(half * nbuf + b, rows[half * nbuf + b],
                       gsA if half == 0 else gsB)

        def body(it, carry):
            process_half(it, 0)
            process_half(it, 1)
            return carry
        lax.fori_loop(0, nit, body, 0)
        plsc.subcore_barrier()

        def ochunk(i, carry):
            k2 = s + i * NS
            pltpu.async_copy(acc_sh.at[pl.ds(k2 * zch, zch)],
                             out_hbm.at[c, pl.ds(k2 * zch, zch)], ssA)
            return carry
        lax.fori_loop(0, cnt, ochunk, 0)
        def odrain(i, carry):
            pltpu.make_async_copy(acc_sh.at[pl.ds(0, zch)],
                                  out_hbm.at[c, pl.ds(0, zch)], ssA).wait()
            return carry
        lax.fori_loop(0, cnt, odrain, 0)

    return k(h2, src, dst2s, w)


def _sc_edge_absdiff(h, src, dst):
    n, din = h.shape
    E = src.shape[0]
    epw = E // NW
    ch = 40
    nch = epw // ch
    nbuf = 5
    nit = nch // (2 * nbuf)
    rem = nch - nit * 2 * nbuf
    assert rem in (0, nbuf)
    nf = din // 16
    mesh = plsc.VectorSubcoreMesh(core_axis_name="c", subcore_axis_name="s",
                                  num_cores=NC, num_subcores=NS)

    scratch = ([pltpu.VMEM((epw,), jnp.int32),
                pltpu.VMEM((epw,), jnp.int32)]
               + [pltpu.VMEM((ch, din), jnp.float32)
                  for _ in range(4 * nbuf)]
               + [pltpu.SemaphoreType.DMA] * 4)

    @functools.partial(
        pl.kernel,
        out_type=jax.ShapeDtypeStruct((E, din), jnp.float32),
        mesh=mesh,
        scratch_types=scratch,
        compiler_params=pltpu.CompilerParams(use_tc_tiling_on_sc=False),
    )
    def k(h_hbm, src_hbm, dst_hbm, e_hbm, src_v, dst_v, *rest):
        hs = rest[:2 * nbuf]
        hd = rest[2 * nbuf:4 * nbuf]
        gsA, gsB, ssA, ssB = rest[4 * nbuf:]
        c = lax.axis_index("c")
        s = lax.axis_index("s")
        wid = c * NS + s
        base = wid * epw
        pltpu.sync_copy(src_hbm.at[pl.ds(base, epw)], src_v)
        pltpu.sync_copy(dst_hbm.at[pl.ds(base, epw)], dst_v)

        def gather2(j, bs, bd, sem):
            pltpu.async_copy(h_hbm.at[src_v.at[pl.ds(j * ch, ch)]], bs, sem)
            pltpu.async_copy(h_hbm.at[dst_v.at[pl.ds(j * ch, ch)]], bd, sem)

        def process(cbase, bss, bds, gsem, ssem, reissue):
            for b in range(nbuf):
                pltpu.make_async_copy(
                    h_hbm.at[src_v.at[pl.ds(0, ch)]], bss[b], gsem).wait()
                pltpu.make_async_copy(
                    h_hbm.at[src_v.at[pl.ds(0, ch)]], bds[b], gsem).wait()
            for b in range(nbuf):
                j = cbase + b
                bs = bss[b]
                bd = bds[b]

                def ediff(e, cc, _bs=bs, _bd=bd):
                    for f in range(nf):
                        sl = pl.ds(f * 16, 16)
                        _bs[e, sl] = jnp.abs(_bs[e, sl] - _bd[e, sl])
                    return cc
                lax.fori_loop(0, ch, ediff, 0)
                pltpu.async_copy(bs, e_hbm.at[pl.ds(base + j * ch, ch)], ssem)
            for b in range(nbuf):
                pltpu.make_async_copy(
                    bss[b], e_hbm.at[pl.ds(base, ch)], ssem).wait()
            if reissue is not None:
                for b in range(nbuf):
                    gather2(reissue + b, bss[b], bds[b], gsem)

        bA, bAd = hs[:nbuf], hd[:nbuf]
        bB, bBd = hs[nbuf:], hd[nbuf:]
        for b in range(nbuf):
            gather2(b, bA[b], bAd[b], gsA)
            gather2(nbuf + b, bB[b], bBd[b], gsB)

        def body(it, carry):
            cb = it * 2 * nbuf
            if rem:
                process(cb, bA, bAd, gsA, ssA, cb + 2 * nbuf)
            else:
                process(cb, bA, bAd, gsA, ssA, None)

                @pl.when(it < nit - 1)
                def _():
                    for b in range(nbuf):
                        gather2(cb + 2 * nbuf + b, bA[b], bAd[b], gsA)
            process(cb + nbuf, bB, bBd, gsB, ssB, None)

            @pl.when(it < nit - 1)
            def _():
                for b in range(nbuf):
                    gather2(cb + 3 * nbuf + b, bB[b], bBd[b], gsB)
            return carry
        lax.fori_loop(0, nit, body, 0)
        if rem:
            process(nit * 2 * nbuf, bA, bAd, gsA, ssA, None)

    return k(h, src, dst)


def kernel(x, edge_index, edge_weight, g_size, emb_W, emb_b, bn_g, bn_b,
           blk1_W1, blk1_b1, blk1_W2, blk1_b2, blk1_g, blk1_bb,
           blk2_W1, blk2_b1, blk2_W2, blk2_b2, blk2_g, blk2_bb,
           blk3_W1, blk3_b1, blk3_W2, blk3_b2, blk3_g, blk3_bb,
           blk4_W1, blk4_b1, blk4_W2, blk4_b2, blk4_g, blk4_bb,
           wc_W1, wc_b1, wc_W2, wc_b2, fc_W, fc_b):
    E = edge_index.shape[1]
    n = x.shape[0]
    src = edge_index[0]
    dst = edge_index[1]
    dst2 = dst.reshape(NW, E // NW // 40, 40)
    dst2s = dst.reshape(NS, E // NS // 40, 40)

    h = _tc_embed(x, emb_W, emb_b, bn_g, bn_b)

    p = _sc_spmm(h, src, dst2, edge_weight)
    h = _tc_block(h, p[0], p[1], blk1_W1, blk1_b1, blk1_W2, blk1_b2,
                  blk1_g, blk1_bb)
    p = _sc_spmm(h, src, dst2, edge_weight)
    h = _tc_block(h, p[0], p[1], blk2_W1, blk2_b1, blk2_W2, blk2_b2,
                  blk2_g, blk2_bb)
    p = _sc_spmm(h, src, dst2, edge_weight)
    h = _tc_block(h, p[0], p[1], blk3_W1, blk3_b1, blk3_W2, blk3_b2,
                  blk3_g, blk3_bb)
    p = _sc_spmm_split(h.reshape(2 * n, 64), src, dst2s, edge_weight, n)
    h, out = _tc_block_final(h, p[0], p[1], blk4_W1, blk4_b1, blk4_W2,
                             blk4_b2, blk4_g, blk4_bb, fc_W, fc_b)

    eh = E // 2
    e0 = _sc_edge_absdiff(h, src[:eh], dst[:eh])
    e1 = _sc_edge_absdiff(h, src[eh:], dst[eh:])
    m0 = _tc_edge_mlp(e0, wc_W1, wc_b1, wc_W2, wc_b2)
    m1 = _tc_edge_mlp(e1, wc_W1, wc_b1, wc_W2, wc_b2)
    wnew = jnp.concatenate([m0, m1])
    return (out, wnew, g_size)


# speedup vs baseline: 1.1331x; 1.0151x over previous
import functools

import jax
import jax.numpy as jnp
from jax import lax
from jax.experimental import pallas as pl
from jax.experimental.pallas import tpu as pltpu
from jax.experimental.pallas import tpu_sc as plsc

NC = 2
NS = 16
NW = NC * NS


def _bn(h, g, b):
    m = jnp.mean(h, axis=0, keepdims=True)
    v = jnp.mean((h - m) * (h - m), axis=0, keepdims=True)
    return (h - m) * lax.rsqrt(v + 1e-5) * g + b


def _embed_body(x_ref, w_ref, b_ref, g_ref, bb_ref, o_ref):
    h = jnp.dot(x_ref[...], w_ref[...], preferred_element_type=jnp.float32)
    h = h + b_ref[...]
    o_ref[...] = jnp.maximum(_bn(h, g_ref[...], bb_ref[...]), 0.0)


def _tc_embed(x, w, b, g, bb):
    n = x.shape[0]
    dout = w.shape[1]
    return pl.pallas_call(
        _embed_body,
        out_shape=jax.ShapeDtypeStruct((n, dout), jnp.float32),
    )(x, w, b.reshape(1, -1), g.reshape(1, -1), bb.reshape(1, -1))


def _block_body(h_ref, p0_ref, p1_ref, w1_ref, b1_ref, w2_ref, b2_ref,
                g_ref, bb_ref, o_ref):
    h = h_ref[...]
    s = p0_ref[...] + p1_ref[...]
    din = h.shape[1]
    w1 = w1_ref[...]
    w2 = w2_ref[...]
    a = (jnp.dot(h, w1[:din], preferred_element_type=jnp.float32)
         + jnp.dot(s, w1[din:], preferred_element_type=jnp.float32)
         + b1_ref[...])
    c = (jnp.dot(h, w2[:din], preferred_element_type=jnp.float32)
         + jnp.dot(s, w2[din:], preferred_element_type=jnp.float32)
         + b2_ref[...])
    hc = jnp.concatenate([jnp.maximum(a, 0.0), c], axis=1)
    o_ref[...] = _bn(hc, g_ref[...], bb_ref[...])


def _tc_block(h, p0, p1, w1, b1, w2, b2, g, bb):
    n = h.shape[0]
    dout = g.shape[0]
    return pl.pallas_call(
        _block_body,
        out_shape=jax.ShapeDtypeStruct((n, dout), jnp.float32),
    )(h, p0, p1, w1, b1.reshape(1, -1), w2, b2.reshape(1, -1),
      g.reshape(1, -1), bb.reshape(1, -1))


def _block_fin_body(h_ref, p0_ref, p1_ref, w1_ref, b1_ref, w2_ref, b2_ref,
                    g_ref, bb_ref, fw_ref, fb_ref, o_ref, of_ref):
    h = h_ref[...]
    s = jnp.concatenate([p0_ref[...], p1_ref[...]], axis=1)
    din = h.shape[1]
    w1 = w1_ref[...]
    w2 = w2_ref[...]
    a = (jnp.dot(h, w1[:din], preferred_element_type=jnp.float32)
         + jnp.dot(s, w1[din:], preferred_element_type=jnp.float32)
         + b1_ref[...])
    c = (jnp.dot(h, w2[:din], preferred_element_type=jnp.float32)
         + jnp.dot(s, w2[din:], preferred_element_type=jnp.float32)
         + b2_ref[...])
    hc = jnp.concatenate([jnp.maximum(a, 0.0), c], axis=1)
    hb = _bn(hc, g_ref[...], bb_ref[...])
    o_ref[...] = hb
    of_ref[...] = jnp.tanh(
        jnp.dot(hb, fw_ref[...], preferred_element_type=jnp.float32)
        + fb_ref[...])


def _tc_block_final(h, p0, p1, w1, b1, w2, b2, g, bb, fw, fb):
    n = h.shape[0]
    dout = g.shape[0]
    return pl.pallas_call(
        _block_fin_body,
        out_shape=(jax.ShapeDtypeStruct((n, dout), jnp.float32),
                   jax.ShapeDtypeStruct((n, fw.shape[1]), jnp.float32)),
    )(h, p0, p1, w1, b1.reshape(1, -1), w2, b2.reshape(1, -1),
      g.reshape(1, -1), bb.reshape(1, -1), fw, fb.reshape(1, -1))


def _edge_mlp_body(e_ref, w1_ref, b1_ref, w2_ref, b2_ref, o_ref):
    t = jnp.maximum(
        jnp.dot(e_ref[...].astype(jnp.float32), w1_ref[...],
                preferred_element_type=jnp.float32)
        + b1_ref[...], 0.0)
    w = jnp.sum(t * w2_ref[...], axis=1) + b2_ref[0, 0]
    o_ref[...] = jax.nn.sigmoid(w).reshape(1, 1, -1)


def _tc_edge_mlp(e, w1, b1, w2, b2):
    E = e.shape[0]
    che = 4000
    nblk = E // che
    out = pl.pallas_call(
        _edge_mlp_body,
        grid=(nblk,),
        in_specs=[
            pl.BlockSpec((che, e.shape[1]), lambda i: (i, 0)),
            pl.BlockSpec(w1.shape, lambda i: (0, 0)),
            pl.BlockSpec((1, w1.shape[1]), lambda i: (0, 0)),
            pl.BlockSpec((1, w1.shape[1]), lambda i: (0, 0)),
            pl.BlockSpec((1, 1), lambda i: (0, 0)),
        ],
        out_specs=pl.BlockSpec((1, 1, che), lambda i: (i, 0, 0)),
        out_shape=jax.ShapeDtypeStruct((nblk, 1, che), jnp.float32),
    )(e, w1, b1.reshape(1, -1), w2.reshape(1, -1), b2.reshape(1, 1))
    return out.reshape(E)


def _sc_spmm(h, src, dst2, w):
    n, din = h.shape
    E = src.shape[0]
    epw = E // NW
    ch = 40
    nch = epw // ch
    nbuf = 5
    nit = nch // (2 * nbuf)
    zch = 80
    nrc = n // zch
    nf = din // 16
    mesh = plsc.VectorSubcoreMesh(core_axis_name="c", subcore_axis_name="s",
                                  num_cores=NC, num_subcores=NS)

    scratch = ([pltpu.VMEM((epw,), jnp.int32),
                pltpu.VMEM((nch, ch), jnp.int32),
                pltpu.VMEM((epw + 8,), jnp.float32)]
               + [pltpu.VMEM((ch, din), jnp.float32) for _ in range(2 * nbuf)]
               + [pltpu.VMEM((zch, din), jnp.float32),
                  pltpu.VMEM_SHARED((n, din), jnp.float32)]
               + [pltpu.SemaphoreType.DMA] * 4)

    @functools.partial(
        pl.kernel,
        out_type=jax.ShapeDtypeStruct((NC, n, din), jnp.float32),
        mesh=mesh,
        scratch_types=scratch,
        compiler_params=pltpu.CompilerParams(use_tc_tiling_on_sc=False),
    )
    def k(h_hbm, src_hbm, dst_hbm, w_hbm, out_hbm, src_v, dst_v, w_v, *rest):
        rows = rest[:2 * nbuf]
        zz_v, acc_sh, gsA, gsB, ssA, ssB = rest[2 * nbuf:]
        c = lax.axis_index("c")
        s = lax.axis_index("s")
        wid = c * NS + s
        base = wid * epw
        pltpu.async_copy(src_hbm.at[pl.ds(base, epw)], src_v, gsA)
        pltpu.async_copy(dst_hbm.at[wid], dst_v, gsA)
        pltpu.async_copy(w_hbm.at[pl.ds(base, epw)], w_v.at[pl.ds(0, epw)],
                         gsA)

        def zrow(r, carry):
            for f in range(nf):
                zz_v[r, pl.ds(f * 16, 16)] = jnp.zeros((16,), jnp.float32)
            return carry
        lax.fori_loop(0, zch, zrow, 0)
        cnt = (nrc - s + NS - 1) // NS
        def zchunk(i, carry):
            pltpu.async_copy(zz_v, acc_sh.at[pl.ds((s + i * NS) * zch, zch)],
                             ssA)
            return carry
        lax.fori_loop(0, cnt, zchunk, 0)
        pltpu.make_async_copy(src_hbm.at[pl.ds(base, epw)], src_v, gsA).wait()
        pltpu.make_async_copy(dst_hbm.at[wid], dst_v, gsA).wait()
        pltpu.make_async_copy(w_hbm.at[pl.ds(base, epw)],
                              w_v.at[pl.ds(0, epw)], gsA).wait()
        def zdrain(i, carry):
            pltpu.make_async_copy(zz_v, acc_sh.at[pl.ds(0, zch)], ssA).wait()
            return carry
        lax.fori_loop(0, cnt, zdrain, 0)
        plsc.subcore_barrier()

        def gather(j, buf, sem):
            pltpu.async_copy(h_hbm.at[src_v.at[pl.ds(j * ch, ch)]], buf, sem)

        def process_half(it, half):
            bufs = rows[half * nbuf:(half + 1) * nbuf]
            gsem = gsA if half == 0 else gsB
            ssem = ssA if half == 0 else ssB
            cbase = it * 2 * nbuf + half * nbuf
            for b in range(nbuf):
                pltpu.make_async_copy(
                    h_hbm.at[src_v.at[pl.ds(0, ch)]], bufs[b], gsem).wait()
            for b in range(nbuf):
                j = cbase + b
                buf = bufs[b]

                def g8(g, cc, _buf=buf, _j=j):
                    wvec = w_v[pl.ds(_j * ch + g * 8, 16)]
                    for e8 in range(8):
                        wb = jnp.full((16,), wvec[e8], jnp.float32)
                        for f in range(nf):
                            _buf[g * 8 + e8, pl.ds(f * 16, 16)] = (
                                _buf[g * 8 + e8, pl.ds(f * 16, 16)] * wb)
                    return cc
                lax.fori_loop(0, ch // 8, g8, 0)
                pltpu.async_copy(buf, acc_sh.at[dst_v.at[j]], ssem, add=True)
            for b in range(nbuf):
                pltpu.make_async_copy(
                    bufs[b], acc_sh.at[dst_v.at[0]], ssem).wait()

            @pl.when(it < nit - 1)
            def _():
                for b in range(nbuf):
                    gather((it + 1) * 2 * nbuf + half * nbuf + b,
                           bufs[b], gsem)

        for half in range(2):
            for b in range(nbuf):
                gather(half * nbuf + b, rows[half * nbuf + b],
                       gsA if half == 0 else gsB)

        def body(it, carry):
            process_half(it, 0)
            process_half(it, 1)
            return carry
        lax.fori_loop(0, nit, body, 0)
        plsc.subcore_barrier()

        def ochunk(i, carry):
            k2 = s + i * NS
            pltpu.async_copy(acc_sh.at[pl.ds(k2 * zch, zch)],
                             out_hbm.at[c, pl.ds(k2 * zch, zch)], ssA)
            return carry
        lax.fori_loop(0, cnt, ochunk, 0)
        def odrain(i, carry):
            pltpu.make_async_copy(acc_sh.at[pl.ds(0, zch)],
                                  out_hbm.at[c, pl.ds(0, zch)], ssA).wait()
            return carry
        lax.fori_loop(0, cnt, odrain, 0)

    return k(h, src, dst2, w)


def _sc_spmm_split(h2, src, dst2s, w, n):
    dh = h2.shape[1]
    E = src.shape[0]
    epc = E // NS
    ch = 40
    nch = epc // ch
    nbuf = 5
    nit = nch // (2 * nbuf)
    zch = 40
    nrc = n // zch
    nf = dh // 16
    mesh = plsc.VectorSubcoreMesh(core_axis_name="c", subcore_axis_name="s",
                                  num_cores=NC, num_subcores=NS)

    scratch = ([pltpu.VMEM((epc,), jnp.int32),
                pltpu.VMEM((nch, ch), jnp.int32),
                pltpu.VMEM((epc + 8,), jnp.float32)]
               + [pltpu.VMEM((ch, dh), jnp.float32) for _ in range(2 * nbuf)]
               + [pltpu.VMEM((zch, dh), jnp.float32),
                  pltpu.VMEM_SHARED((n, dh), jnp.float32)]
               + [pltpu.SemaphoreType.DMA] * 4)

    @functools.partial(
        pl.kernel,
        out_type=jax.ShapeDtypeStruct((NC, n, dh), jnp.float32),
        mesh=mesh,
        scratch_types=scratch,
        compiler_params=pltpu.CompilerParams(use_tc_tiling_on_sc=False),
    )
    def k(h_hbm, src_hbm, dst_hbm, w_hbm, out_hbm, src_v, dst_v, w_v, *rest):
        rows = rest[:2 * nbuf]
        zz_v, acc_sh, gsA, gsB, ssA, ssB = rest[2 * nbuf:]
        c = lax.axis_index("c")
        s = lax.axis_index("s")
        base = s * epc
        pltpu.async_copy(src_hbm.at[pl.ds(base, epc)], src_v, gsA)
        pltpu.async_copy(dst_hbm.at[s], dst_v, gsA)
        pltpu.async_copy(w_hbm.at[pl.ds(base, epc)], w_v.at[pl.ds(0, epc)],
                         gsA)

        def zrow(r, cc):
            for f in range(nf):
                zz_v[r, pl.ds(f * 16, 16)] = jnp.zeros((16,), jnp.float32)
            return cc
        lax.fori_loop(0, zch, zrow, 0)
        cnt = (nrc - s + NS - 1) // NS
        def zchunk(i, cc):
            pltpu.async_copy(zz_v, acc_sh.at[pl.ds((s + i * NS) * zch, zch)],
                             ssA)
            return cc
        lax.fori_loop(0, cnt, zchunk, 0)
        pltpu.make_async_copy(src_hbm.at[pl.ds(base, epc)], src_v, gsA).wait()
        pltpu.make_async_copy(dst_hbm.at[s], dst_v, gsA).wait()
        pltpu.make_async_copy(w_hbm.at[pl.ds(base, epc)],
                              w_v.at[pl.ds(0, epc)], gsA).wait()

        cvec = jnp.full((16,), c, jnp.int32)

        def ixf(g, cc):
            sl = pl.ds(g * 16, 16)
            v = src_v[sl]
            src_v[sl] = v + v + cvec
            return cc
        lax.fori_loop(0, epc // 16, ixf, 0)

        def zdrain(i, cc):
            pltpu.make_async_copy(zz_v, acc_sh.at[pl.ds(0, zch)], ssA).wait()
            return cc
        lax.fori_loop(0, cnt, zdrain, 0)
        plsc.subcore_barrier()

        def gather(j, buf, sem):
            pltpu.async_copy(h_hbm.at[src_v.at[pl.ds(j * ch, ch)]], buf, sem)

        def process_half(it, half):
            bufs = rows[half * nbuf:(half + 1) * nbuf]
            gsem = gsA if half == 0 else gsB
            ssem = ssA if half == 0 else ssB
            cbase = it * 2 * nbuf + half * nbuf
            for b in range(nbuf):
                pltpu.make_async_copy(
                    h_hbm.at[src_v.at[pl.ds(0, ch)]], bufs[b], gsem).wait()
            for b in range(nbuf):
                j = cbase + b
                buf = bufs[b]

                def g8(g, cc, _buf=buf, _j=j):
                    wvec = w_v[pl.ds(_j * ch + g * 8, 16)]
                    for e8 in range(8):
                        wb = jnp.full((16,), wvec[e8], jnp.float32)
                        for f in range(nf):
                            _buf[g * 8 + e8, pl.ds(f * 16, 16)] = (
                                _buf[g * 8 + e8, pl.ds(f * 16, 16)] * wb)
                    return cc
                lax.fori_loop(0, ch // 8, g8, 0)
                pltpu.async_copy(buf, acc_sh.at[dst_v.at[j]], ssem, add=True)
            for b in range(nbuf):
                pltpu.make_async_copy(
                    bufs[b], acc_sh.at[dst_v.at[0]], ssem).wait()

            @pl.when(it < nit - 1)
            def _():
                for b in range(nbuf):
                    gather((it + 1) * 2 * nbuf + half * nbuf + b,
                           bufs[b], gsem)

        for half in range(2):
            for b in range(nbuf):
                gather(half * nbuf + b, rows[half * nbuf + b],
                       gsA if half == 0 else gsB)

        def body(it, carry):
            process_half(it, 0)
            process_half(it, 1)
            return carry
        lax.fori_loop(0, nit, body, 0)
        plsc.subcore_barrier()

        def ochunk(i, carry):
            k2 = s + i * NS
            pltpu.async_copy(acc_sh.at[pl.ds(k2 * zch, zch)],
                             out_hbm.at[c, pl.ds(k2 * zch, zch)], ssA)
            return carry
        lax.fori_loop(0, cnt, ochunk, 0)
        def odrain(i, carry):
            pltpu.make_async_copy(acc_sh.at[pl.ds(0, zch)],
                                  out_hbm.at[c, pl.ds(0, zch)], ssA).wait()
            return carry
        lax.fori_loop(0, cnt, odrain, 0)

    return k(h2, src, dst2s, w)


def _sc_edge_absdiff(h, src, dst):
    n, din = h.shape
    E = src.shape[0]
    epw = E // NW
    ch = 40
    nch = epw // ch
    nbuf = 5
    nit = nch // (2 * nbuf)
    rem = nch - nit * 2 * nbuf
    assert rem in (0, nbuf)
    nf = din // 16
    mesh = plsc.VectorSubcoreMesh(core_axis_name="c", subcore_axis_name="s",
                                  num_cores=NC, num_subcores=NS)

    scratch = ([pltpu.VMEM((epw,), jnp.int32),
                pltpu.VMEM((epw,), jnp.int32)]
               + [pltpu.VMEM((ch, din), jnp.float32)
                  for _ in range(4 * nbuf)]
               + [pltpu.SemaphoreType.DMA] * 4)

    @functools.partial(
        pl.kernel,
        out_type=jax.ShapeDtypeStruct((E, din), jnp.float32),
        mesh=mesh,
        scratch_types=scratch,
        compiler_params=pltpu.CompilerParams(use_tc_tiling_on_sc=False),
    )
    def k(h_hbm, src_hbm, dst_hbm, e_hbm, src_v, dst_v, *rest):
        hs = rest[:2 * nbuf]
        hd = rest[2 * nbuf:4 * nbuf]
        gsA, gsB, ssA, ssB = rest[4 * nbuf:]
        c = lax.axis_index("c")
        s = lax.axis_index("s")
        wid = c * NS + s
        base = wid * epw
        pltpu.sync_copy(src_hbm.at[pl.ds(base, epw)], src_v)
        pltpu.sync_copy(dst_hbm.at[pl.ds(base, epw)], dst_v)

        def gather2(j, bs, bd, sem):
            pltpu.async_copy(h_hbm.at[src_v.at[pl.ds(j * ch, ch)]], bs, sem)
            pltpu.async_copy(h_hbm.at[dst_v.at[pl.ds(j * ch, ch)]], bd, sem)

        def process(cbase, bss, bds, gsem, ssem, reissue):
            for b in range(nbuf):
                pltpu.make_async_copy(
                    h_hbm.at[src_v.at[pl.ds(0, ch)]], bss[b], gsem).wait()
                pltpu.make_async_copy(
                    h_hbm.at[src_v.at[pl.ds(0, ch)]], bds[b], gsem).wait()
            for b in range(nbuf):
                j = cbase + b
                bs = bss[b]
                bd = bds[b]

                def ediff(e, cc, _bs=bs, _bd=bd):
                    for f in range(nf):
                        sl = pl.ds(f * 16, 16)
                        _bs[e, sl] = jnp.abs(_bs[e, sl] - _bd[e, sl])
                    return cc
                lax.fori_loop(0, ch, ediff, 0)
                pltpu.async_copy(bs, e_hbm.at[pl.ds(base + j * ch, ch)], ssem)
            for b in range(nbuf):
                pltpu.make_async_copy(
                    bss[b], e_hbm.at[pl.ds(base, ch)], ssem).wait()
            if reissue is not None:
                for b in range(nbuf):
                    gather2(reissue + b, bss[b], bds[b], gsem)

        bA, bAd = hs[:nbuf], hd[:nbuf]
        bB, bBd = hs[nbuf:], hd[nbuf:]
        for b in range(nbuf):
            gather2(b, bA[b], bAd[b], gsA)
            gather2(nbuf + b, bB[b], bBd[b], gsB)

        def body(it, carry):
            cb = it * 2 * nbuf
            if rem:
                process(cb, bA, bAd, gsA, ssA, cb + 2 * nbuf)
            else:
                process(cb, bA, bAd, gsA, ssA, None)

                @pl.when(it < nit - 1)
                def _():
                    for b in range(nbuf):
                        gather2(cb + 2 * nbuf + b, bA[b], bAd[b], gsA)
            process(cb + nbuf, bB, bBd, gsB, ssB, None)

            @pl.when(it < nit - 1)
            def _():
                for b in range(nbuf):
                    gather2(cb + 3 * nbuf + b, bB[b], bBd[b], gsB)
            return carry
        lax.fori_loop(0, nit, body, 0)
        if rem:
            process(nit * 2 * nbuf, bA, bAd, gsA, ssA, None)

    return k(h, src, dst)


def kernel(x, edge_index, edge_weight, g_size, emb_W, emb_b, bn_g, bn_b,
           blk1_W1, blk1_b1, blk1_W2, blk1_b2, blk1_g, blk1_bb,
           blk2_W1, blk2_b1, blk2_W2, blk2_b2, blk2_g, blk2_bb,
           blk3_W1, blk3_b1, blk3_W2, blk3_b2, blk3_g, blk3_bb,
           blk4_W1, blk4_b1, blk4_W2, blk4_b2, blk4_g, blk4_bb,
           wc_W1, wc_b1, wc_W2, wc_b2, fc_W, fc_b):
    E = edge_index.shape[1]
    n = x.shape[0]
    src = edge_index[0]
    dst = edge_index[1]
    dst2 = dst.reshape(NW, E // NW // 40, 40)
    dst2s = dst.reshape(NS, E // NS // 40, 40)

    h = _tc_embed(x, emb_W, emb_b, bn_g, bn_b)

    p = _sc_spmm(h, src, dst2, edge_weight)
    h = _tc_block(h, p[0], p[1], blk1_W1, blk1_b1, blk1_W2, blk1_b2,
                  blk1_g, blk1_bb)
    p = _sc_spmm(h, src, dst2, edge_weight)
    h = _tc_block(h, p[0], p[1], blk2_W1, blk2_b1, blk2_W2, blk2_b2,
                  blk2_g, blk2_bb)
    p = _sc_spmm(h, src, dst2, edge_weight)
    h = _tc_block(h, p[0], p[1], blk3_W1, blk3_b1, blk3_W2, blk3_b2,
                  blk3_g, blk3_bb)
    p = _sc_spmm_split(h.reshape(2 * n, 64), src, dst2s, edge_weight, n)
    h, out = _tc_block_final(h, p[0], p[1], blk4_W1, blk4_b1, blk4_W2,
                             blk4_b2, blk4_g, blk4_bb, fc_W, fc_b)

    sizes = [3 * E // 10, 3 * E // 10, E // 5, E // 5]
    es = []
    lo = 0
    for sz in sizes:
        es.append(_sc_edge_absdiff(h, src[lo:lo + sz], dst[lo:lo + sz]))
        lo += sz
    ms = [_tc_edge_mlp(e_i, wc_W1, wc_b1, wc_W2, wc_b2) for e_i in es]
    wnew = jnp.concatenate(ms)
    return (out, wnew, g_size)
